# baseline (speedup 1.0000x reference)
import jax
import jax.numpy as jnp
from jax import lax
from jax.experimental import pallas as pl
from jax.experimental.pallas import tpu as pltpu
from jax.experimental.pallas import tpu_sc as plsc

_B = 16384
_D = 32
_NC = 2
_NS = 16
_NW = _NC * _NS
_BPW = _B // _NW
_CH = 128
_NCH = _BPW // _CH


def _body(table_hbm, src_hbm, dst_hbm, out_hbm,
          sidx_v, didx_v, srows_v, drows_v, out_v, sem):
    wid = lax.axis_index("s") * _NC + lax.axis_index("c")
    base = wid * _BPW

    pltpu.sync_copy(src_hbm.at[pl.ds(base, _BPW)], sidx_v)
    pltpu.sync_copy(dst_hbm.at[pl.ds(base, _BPW)], didx_v)

    copies = []
    for j in range(_NCH):
        sl = pl.ds(j * _CH, _CH)
        copies.append(
            pltpu.async_copy(table_hbm.at[sidx_v.at[sl]], srows_v.at[sl], sem))
        copies.append(
            pltpu.async_copy(table_hbm.at[didx_v.at[sl]], drows_v.at[sl], sem))
    for c in copies:
        c.wait()

    lanes = lax.iota(jnp.int32, 16)
    perm = {h: lanes ^ h for h in (1, 2, 4, 8)}
    bit = {h: (lanes & h) != 0 for h in (1, 2, 4, 8)}

    def _take(x, idx):
        return jnp.take_along_axis(x, idx, axis=0, mode="promise_in_bounds")

    def block(b, carry):
        regs = []
        for r_local in range(16):
            r = b * 16 + r_local
            lo = srows_v[r, pl.ds(0, 16)] * drows_v[r, pl.ds(0, 16)]
            hi = srows_v[r, pl.ds(16, 16)] * drows_v[r, pl.ds(16, 16)]
            regs.append(lo + hi)
        for h in (1, 2, 4, 8):
            nxt = []
            for i in range(0, len(regs), 2):
                u, v = regs[i], regs[i + 1]
                t1 = jnp.where(bit[h], v, u)
                t2 = _take(jnp.where(bit[h], u, v), perm[h])
                nxt.append(t1 + t2)
            regs = nxt
        acc = regs[0]
        out_v[pl.ds(b * 16, 16)] = 1.0 / (1.0 + jnp.exp(-acc))
        return carry

    lax.fori_loop(0, _BPW // 16, block, 0)

    pltpu.sync_copy(out_v, out_hbm.at[pl.ds(base, _BPW)])


def kernel(embedding_user, src, dst):
    mesh = plsc.VectorSubcoreMesh(core_axis_name="c", subcore_axis_name="s")
    k = pl.kernel(
        _body,
        mesh=mesh,
        compiler_params=pltpu.CompilerParams(use_tc_tiling_on_sc=False),
        out_type=jax.ShapeDtypeStruct((_B,), jnp.float32),
        scratch_types=[
            pltpu.VMEM((_BPW,), jnp.int32),
            pltpu.VMEM((_BPW,), jnp.int32),
            pltpu.VMEM((_BPW, _D), jnp.float32),
            pltpu.VMEM((_BPW, _D), jnp.float32),
            pltpu.VMEM((_BPW,), jnp.float32),
            pltpu.SemaphoreType.DMA,
        ],
    )
    return k(embedding_user, src, dst)


# speedup vs baseline: 1.5581x; 1.5581x over previous
import jax
import jax.numpy as jnp
from jax import lax
from jax.experimental import pallas as pl
from jax.experimental.pallas import tpu as pltpu
from jax.experimental.pallas import tpu_sc as plsc

_B = 16384
_D = 32
_NC = 2
_NS = 16
_NW = _NC * _NS
_BPW = _B // _NW
_CH = 16


def _body(table_hbm, src_hbm, dst_hbm, out_hbm,
          sidx_v, didx_v, srows_v, drows_v, out_v, sem):
    wid = lax.axis_index("s") * _NC + lax.axis_index("c")
    base = wid * _BPW

    pltpu.sync_copy(src_hbm.at[pl.ds(base, _BPW)], sidx_v)
    pltpu.sync_copy(dst_hbm.at[pl.ds(base, _BPW)], didx_v)

    lanes = lax.iota(jnp.int32, 16)
    perm = {h: lanes ^ h for h in (1, 2, 4, 8)}
    bit = {h: (lanes & h) != 0 for h in (1, 2, 4, 8)}

    def _take(x, idx):
        return jnp.take_along_axis(x, idx, axis=0, mode="promise_in_bounds")

    def chunk(c, carry):
        sv = sidx_v[pl.ds(c * _CH, _CH)]
        dv = didx_v[pl.ds(c * _CH, _CH)]
        copies = []
        for r in range(_CH):
            copies.append(pltpu.async_copy(
                table_hbm.at[pl.ds(sv[r], 1)], srows_v.at[pl.ds(r, 1)], sem))
            copies.append(pltpu.async_copy(
                table_hbm.at[pl.ds(dv[r], 1)], drows_v.at[pl.ds(r, 1)], sem))
        for cp in copies:
            cp.wait()
        regs = []
        for r in range(_CH):
            lo = srows_v[r, pl.ds(0, 16)] * drows_v[r, pl.ds(0, 16)]
            hi = srows_v[r, pl.ds(16, 16)] * drows_v[r, pl.ds(16, 16)]
            regs.append(lo + hi)
        for h in (1, 2, 4, 8):
            nxt = []
            for i in range(0, len(regs), 2):
                u, v = regs[i], regs[i + 1]
                t1 = jnp.where(bit[h], v, u)
                t2 = _take(jnp.where(bit[h], u, v), perm[h])
                nxt.append(t1 + t2)
            regs = nxt
        acc = regs[0]
        out_v[pl.ds(c * _CH, _CH)] = 1.0 / (1.0 + jnp.exp(-acc))
        return carry

    lax.fori_loop(0, _BPW // _CH, chunk, 0)

    pltpu.sync_copy(out_v, out_hbm.at[pl.ds(base, _BPW)])


def kernel(embedding_user, src, dst):
    mesh = plsc.VectorSubcoreMesh(core_axis_name="c", subcore_axis_name="s")
    k = pl.kernel(
        _body,
        mesh=mesh,
        out_type=jax.ShapeDtypeStruct((_B,), jnp.float32),
        scratch_types=[
            pltpu.VMEM((_BPW,), jnp.int32),
            pltpu.VMEM((_BPW,), jnp.int32),
            pltpu.VMEM((_CH, _D), jnp.float32),
            pltpu.VMEM((_CH, _D), jnp.float32),
            pltpu.VMEM((_BPW,), jnp.float32),
            pltpu.SemaphoreType.DMA,
        ],
    )
    return k(embedding_user, src, dst)
